# baseline (speedup 1.0000x reference)
import functools

import jax
import jax.numpy as jnp
from jax import lax
from jax.experimental import pallas as pl
from jax.experimental.pallas import tpu as pltpu

_GATHER_M = 32


def _node_transform_kernel(h_ref, w1s_ref, w1d_ref, b1_ref, pq_ref):
    hb = h_ref[...].astype(jnp.bfloat16)
    d = h_ref.shape[1]
    p = jnp.dot(hb, w1s_ref[...], preferred_element_type=jnp.float32) + b1_ref[...]
    q = jnp.dot(hb, w1d_ref[...], preferred_element_type=jnp.float32)
    pq_ref[:, :d] = p.astype(jnp.bfloat16)
    pq_ref[:, d:] = q.astype(jnp.bfloat16)


def _edge_gather_score_kernel(pq_ref, src_ref, dst_ref, w2_ref, b2_ref,
                              out_ref, tsa_ref, tda_ref, tsb_ref, tdb_ref,
                              *, tile_e):
    m = _GATHER_M
    di = pq_ref.shape[2] // 2

    def gather(base, ts, td):
        for mi in range(m):
            s_i = src_ref[0, 0, base + mi]
            d_i = dst_ref[0, 0, base + mi]
            ts[mi] = pq_ref[s_i]
            td[mi] = pq_ref[d_i]

    def score(base, ts, td):
        a = pltpu.bitcast(ts[...], jnp.bfloat16)[:, :, :di]
        b = pltpu.bitcast(td[...], jnp.bfloat16)[:, :, di:]
        x = jnp.maximum((a + b).astype(jnp.float32), 0.0)
        y = x * w2_ref[...]
        s = jnp.sum(jnp.sum(y, axis=1), axis=1, keepdims=True) + b2_ref[0, 0]
        out_ref[pl.ds(base, m), :] = s

    n_chunks = tile_e // m

    def body(k, carry):
        b0 = pl.multiple_of(2 * k * m, m)
        b1 = pl.multiple_of(b0 + m, m)
        b2 = jnp.minimum(b1 + m, tile_e - m)
        gather(b1, tsb_ref, tdb_ref)
        score(b0, tsa_ref, tda_ref)
        gather(b2, tsa_ref, tda_ref)
        score(b1, tsb_ref, tdb_ref)
        return carry

    gather(0, tsa_ref, tda_ref)
    lax.fori_loop(0, n_chunks // 2, body, 0)


def _round_up(x, m):
    return ((x + m - 1) // m) * m


def kernel(w1, b1, w2, b2, src, dst, h):
    N, D = int(h.shape[0]), int(h.shape[1])
    E = int(src.shape[0])

    w1b = w1.astype(jnp.bfloat16)
    w1s, w1d = w1b[:D], w1b[D:]
    b1r = b1.reshape(1, D).astype(jnp.float32)

    tile_n = 2000 if N % 2000 == 0 else 2048
    n_pad = _round_up(N, tile_n)
    hp = h if n_pad == N else jnp.pad(h, ((0, n_pad - N), (0, 0)))
    pq = pl.pallas_call(
        _node_transform_kernel,
        out_shape=jax.ShapeDtypeStruct((n_pad, 2 * D), jnp.bfloat16),
        grid=(n_pad // tile_n,),
        in_specs=[
            pl.BlockSpec((tile_n, D), lambda i: (i, 0)),
            pl.BlockSpec((D, D), lambda i: (0, 0)),
            pl.BlockSpec((D, D), lambda i: (0, 0)),
            pl.BlockSpec((1, D), lambda i: (0, 0)),
        ],
        out_specs=pl.BlockSpec((tile_n, 2 * D), lambda i: (i, 0)),
        compiler_params=pltpu.CompilerParams(dimension_semantics=("parallel",)),
    )(hp, w1s, w1d, b1r)

    d_i32 = D
    pq_i32 = lax.bitcast_convert_type(pq.reshape(n_pad, d_i32, 2), jnp.int32)
    pq_i32 = pq_i32.reshape(n_pad, 1, d_i32)

    tile_e = 2048
    e_pad = _round_up(E, 2 * tile_e)
    if e_pad != E:
        pad = e_pad - E
        src = jnp.concatenate([src, jnp.zeros((pad,), src.dtype)])
        dst = jnp.concatenate([dst, jnp.zeros((pad,), dst.dtype)])
    num_tiles = e_pad // tile_e
    half = num_tiles // 2
    src3 = src.reshape(num_tiles, 1, tile_e)
    dst3 = dst.reshape(num_tiles, 1, tile_e)

    w2bc = jnp.broadcast_to(
        w2.reshape(D // 2, 2).T.reshape(1, 2, D // 2), (_GATHER_M, 2, D // 2)
    ).astype(jnp.float32)
    b2r = b2.reshape(1, 1).astype(jnp.float32)

    smem = pltpu.MemorySpace.SMEM
    slab = pltpu.VMEM((_GATHER_M, 1, d_i32), jnp.int32)
    out = pl.pallas_call(
        functools.partial(_edge_gather_score_kernel, tile_e=tile_e),
        out_shape=jax.ShapeDtypeStruct((e_pad, 1), jnp.float32),
        grid=(2, half),
        in_specs=[
            pl.BlockSpec((n_pad, 1, d_i32), lambda c, i: (0, 0, 0)),
            pl.BlockSpec((1, 1, tile_e), lambda c, i: (c * half + i, 0, 0),
                         memory_space=smem),
            pl.BlockSpec((1, 1, tile_e), lambda c, i: (c * half + i, 0, 0),
                         memory_space=smem),
            pl.BlockSpec((_GATHER_M, 2, D // 2), lambda c, i: (0, 0, 0)),
            pl.BlockSpec(memory_space=smem),
        ],
        out_specs=pl.BlockSpec((tile_e, 1), lambda c, i: (c * half + i, 0)),
        scratch_shapes=[slab, slab, slab, slab],
        compiler_params=pltpu.CompilerParams(
            dimension_semantics=("parallel", "arbitrary")),
    )(pq_i32, src3, dst3, w2bc, b2r)
    return out[:E, 0]


# speedup vs baseline: 1.0990x; 1.0990x over previous
import functools

import jax
import jax.numpy as jnp
from jax import lax
from jax.experimental import pallas as pl
from jax.experimental.pallas import tpu as pltpu

_GATHER_M = 128


def _node_transform_kernel(h_ref, w1s_ref, w1d_ref, b1_ref, pq_ref):
    hb = h_ref[...].astype(jnp.bfloat16)
    d = h_ref.shape[1]
    p = jnp.dot(hb, w1s_ref[...], preferred_element_type=jnp.float32) + b1_ref[...]
    q = jnp.dot(hb, w1d_ref[...], preferred_element_type=jnp.float32)
    pq_ref[:, :d] = p.astype(jnp.bfloat16)
    pq_ref[:, d:] = q.astype(jnp.bfloat16)


def _edge_gather_score_kernel(pq_ref, src_ref, dst_ref, w2_ref,
                              b2_ref, out_ref, tsa_ref, tda_ref, tsb_ref,
                              tdb_ref, *, tile_e):
    m = _GATHER_M
    di = pq_ref.shape[2] // 2

    def gather(base, ts, td):
        for mi in range(m):
            s_i = src_ref[0, 0, base + mi]
            d_i = dst_ref[0, 0, base + mi]
            ts[mi] = pq_ref[s_i]
            td[mi] = pq_ref[d_i]

    def score(base, ts, td):
        a = pltpu.bitcast(ts[:, :, 0:di], jnp.bfloat16)
        b = pltpu.bitcast(td[:, :, di:2 * di], jnp.bfloat16)
        x = jnp.maximum((a + b).astype(jnp.float32), 0.0)
        y = x * w2_ref[...]
        s = jnp.sum(jnp.sum(y, axis=1), axis=1, keepdims=True) + b2_ref[0, 0]
        out_ref[pl.ds(base, m), :] = s

    n_chunks = tile_e // m

    def body(k, carry):
        b0 = pl.multiple_of(2 * k * m, m)
        b1 = pl.multiple_of(b0 + m, m)
        b2 = jnp.minimum(b1 + m, tile_e - m)
        gather(b1, tsb_ref, tdb_ref)
        score(b0, tsa_ref, tda_ref)
        gather(b2, tsa_ref, tda_ref)
        score(b1, tsb_ref, tdb_ref)
        return carry

    gather(0, tsa_ref, tda_ref)
    lax.fori_loop(0, n_chunks // 2, body, 0)


def _round_up(x, m):
    return ((x + m - 1) // m) * m


def kernel(w1, b1, w2, b2, src, dst, h):
    N, D = int(h.shape[0]), int(h.shape[1])
    E = int(src.shape[0])

    w1b = w1.astype(jnp.bfloat16)
    w1s, w1d = w1b[:D], w1b[D:]
    b1r = b1.reshape(1, D).astype(jnp.float32)

    tile_n = 2000 if N % 2000 == 0 else 2048
    n_pad = _round_up(N, tile_n)
    hp = h if n_pad == N else jnp.pad(h, ((0, n_pad - N), (0, 0)))
    pq = pl.pallas_call(
        _node_transform_kernel,
        out_shape=jax.ShapeDtypeStruct((n_pad, 2 * D), jnp.bfloat16),
        grid=(n_pad // tile_n,),
        in_specs=[
            pl.BlockSpec((tile_n, D), lambda i: (i, 0)),
            pl.BlockSpec((D, D), lambda i: (0, 0)),
            pl.BlockSpec((D, D), lambda i: (0, 0)),
            pl.BlockSpec((1, D), lambda i: (0, 0)),
        ],
        out_specs=pl.BlockSpec((tile_n, 2 * D), lambda i: (i, 0)),
        compiler_params=pltpu.CompilerParams(dimension_semantics=("parallel",)),
    )(hp, w1s, w1d, b1r)

    d_i32 = D
    pq_i32 = lax.bitcast_convert_type(pq.reshape(n_pad, d_i32, 2), jnp.int32)
    pq_i32 = pq_i32.reshape(n_pad, 1, d_i32)

    tile_e = 4096
    e_pad = _round_up(E, 2 * tile_e)
    if e_pad != E:
        pad = e_pad - E
        src = jnp.concatenate([src, jnp.zeros((pad,), src.dtype)])
        dst = jnp.concatenate([dst, jnp.zeros((pad,), dst.dtype)])
    num_tiles = e_pad // tile_e
    half = num_tiles // 2
    src3 = src.reshape(num_tiles, 1, tile_e)
    dst3 = dst.reshape(num_tiles, 1, tile_e)

    w2bc = jnp.broadcast_to(
        w2.reshape(D // 2, 2).T.reshape(1, 2, D // 2), (_GATHER_M, 2, D // 2)
    ).astype(jnp.float32)
    b2r = b2.reshape(1, 1).astype(jnp.float32)

    smem = pltpu.MemorySpace.SMEM
    slab = pltpu.VMEM((_GATHER_M, 1, d_i32), jnp.int32)
    out = pl.pallas_call(
        functools.partial(_edge_gather_score_kernel, tile_e=tile_e),
        out_shape=jax.ShapeDtypeStruct((e_pad, 1), jnp.float32),
        grid=(2, half),
        in_specs=[
            pl.BlockSpec((n_pad, 1, d_i32), lambda c, i: (0, 0, 0)),
            pl.BlockSpec((1, 1, tile_e), lambda c, i: (c * half + i, 0, 0),
                         memory_space=smem),
            pl.BlockSpec((1, 1, tile_e), lambda c, i: (c * half + i, 0, 0),
                         memory_space=smem),
            pl.BlockSpec((_GATHER_M, 2, D // 2), lambda c, i: (0, 0, 0)),
            pl.BlockSpec(memory_space=smem),
        ],
        out_specs=pl.BlockSpec((tile_e, 1), lambda c, i: (c * half + i, 0)),
        scratch_shapes=[slab, slab, slab, slab],
        compiler_params=pltpu.CompilerParams(
            dimension_semantics=("parallel", "arbitrary")),
    )(pq_i32, src3, dst3, w2bc, b2r)
    return out[:E, 0]
